# baseline (speedup 1.0000x reference)
import functools

import jax
import jax.numpy as jnp
from jax import lax
from jax.experimental import pallas as pl
from jax.experimental.pallas import tpu as pltpu
from jax.experimental.pallas import tpu_sc as plsc

H = 12
MBLK = 256
_INTERPRET = False


def _qkv_proj_kernel(x_ref, w_ref, b_ref, o_ref):
    x = x_ref[...]
    for j in range(w_ref.shape[0]):
        o_ref[j] = (
            jnp.dot(x, w_ref[j], preferred_element_type=jnp.float32) + b_ref[j]
        )


def _attn_kernel(q_ref, k_ref, v_ref, o_ref, *, scale):
    q = q_ref[0]
    k = k_ref[0]
    v = v_ref[0]
    s = lax.dot_general(
        q, k, (((1,), (1,)), ((), ())), preferred_element_type=jnp.float32
    ) * scale
    m = jnp.max(s, axis=-1, keepdims=True)
    p = jnp.exp(s - m)
    l = jnp.sum(p, axis=-1, keepdims=True)
    o_ref[0] = jnp.dot(p, v, preferred_element_type=jnp.float32) / l


def _ln(y, g, b):
    mu = jnp.mean(y, axis=-1, keepdims=True)
    d = y - mu
    var = jnp.mean(d * d, axis=-1, keepdims=True)
    return d * lax.rsqrt(var + 1e-5) * g + b


def _oproj_ln_kernel(a_ref, w_ref, b_ref, res_ref, g_ref, be_ref, out_ref):
    y = b_ref[...] + res_ref[...]
    for h in range(w_ref.shape[0]):
        y = y + jnp.dot(a_ref[h], w_ref[h], preferred_element_type=jnp.float32)
    out_ref[...] = _ln(y, g_ref[...], be_ref[...])


def _router_kernel(x_ref, wg_ref, bg_ref, probs_ref, idx_ref, usage_ref,
                   *, n_exp):
    i = pl.program_id(0)
    scores = (
        jnp.dot(x_ref[...], wg_ref[...], preferred_element_type=jnp.float32)
        + bg_ref[...]
    )
    cols = lax.broadcasted_iota(jnp.int32, scores.shape, 1)
    m1 = jnp.max(scores, axis=-1, keepdims=True)
    a1 = jnp.min(jnp.where(scores == m1, cols, n_exp), axis=-1, keepdims=True)
    masked = jnp.where(cols == a1, -jnp.inf, scores)
    m2 = jnp.max(masked, axis=-1, keepdims=True)
    a2 = jnp.min(jnp.where(masked == m2, cols, n_exp), axis=-1, keepdims=True)
    e2 = jnp.exp(m2 - m1)
    p1 = 1.0 / (1.0 + e2)
    p2 = e2 * p1
    probs_ref[...] = jnp.concatenate([p1, p2], axis=1)
    idx_ref[...] = jnp.concatenate(
        [a1.reshape(1, -1, 1), a2.reshape(1, -1, 1)], axis=0
    )
    gw = jnp.where(cols == a1, p1, 0.0) + jnp.where(cols == a2, p2, 0.0)

    @pl.when(i == 0)
    def _():
        usage_ref[...] = jnp.zeros_like(usage_ref)

    usage_ref[...] += jnp.sum(gw, axis=0, keepdims=True)


def _rank_kernel(idx_ref, rank_ref, counts_ref, carry_ref, *, n_exp, nrb, sblk):
    k = pl.program_id(0)
    i = pl.program_id(1)

    @pl.when(jnp.logical_and(k == 0, i == 0))
    def _():
        carry_ref[...] = jnp.zeros_like(carry_ref)

    e_col = idx_ref[0]
    cols = lax.broadcasted_iota(jnp.int32, (sblk, n_exp), 1)
    oh = (e_col == cols).astype(jnp.float32)
    r0 = lax.broadcasted_iota(jnp.int32, (sblk, sblk), 0)
    r1 = lax.broadcasted_iota(jnp.int32, (sblk, sblk), 1)
    strict = (r0 > r1).astype(jnp.float32)
    rank_in = jnp.dot(strict, oh, preferred_element_type=jnp.float32)
    rank = jnp.sum(oh * (rank_in + carry_ref[...]), axis=1, keepdims=True)
    rank_ref[0] = rank.astype(jnp.int32)
    carry_ref[...] += jnp.sum(oh, axis=0, keepdims=True)

    @pl.when(jnp.logical_and(k == 1, i == nrb - 1))
    def _():
        counts_ref[...] = jnp.concatenate(
            [carry_ref[...].astype(jnp.int32),
             jnp.zeros((1, 16 - n_exp), jnp.int32)], axis=1)


def _group_ffn_kernel(be_ref, xs_ref, w1_ref, b1_ref, w2_ref, b2_ref, ys_ref):
    h = jnp.maximum(
        jnp.dot(xs_ref[...], w1_ref[0], preferred_element_type=jnp.float32)
        + b1_ref[0],
        0.0,
    )
    ys_ref[...] = (
        jnp.dot(h, w2_ref[0], preferred_element_type=jnp.float32) + b2_ref[0]
    )


def _combine_ln_kernel(x_ref, ys_ref, p_ref, g_ref, be_ref, out_ref):
    p = p_ref[...]
    y = x_ref[...] + p[:, 0:1] * ys_ref[0] + p[:, 1:2] * ys_ref[1]
    out_ref[...] = _ln(y, g_ref[...], be_ref[...])


def _aux_kernel(u_ref, aux_ref, *, n_layers):
    u = u_ref[...]
    p = u / jnp.sum(u, axis=-1, keepdims=True)
    ent = -jnp.sum(p * jnp.log(p + 1e-9), axis=-1)
    aux_ref[...] = (jnp.sum(ent) / n_layers).reshape(1, 1)


def _sc_route(idx_flat, rank_flat, counts16, n_exp, rp, mblk):
    s2 = idx_flat.shape[0]
    shift = mblk.bit_length() - 1
    mesh = plsc.VectorSubcoreMesh(core_axis_name="c", subcore_axis_name="s")

    @functools.partial(
        pl.kernel,
        out_type=[
            jax.ShapeDtypeStruct((rp,), jnp.int32),
            jax.ShapeDtypeStruct((s2,), jnp.int32),
            jax.ShapeDtypeStruct((32,), jnp.int32),
        ],
        mesh=mesh,
        compiler_params=pltpu.CompilerParams(needs_layout_passes=False),
        scratch_types=[
            pltpu.VMEM((s2,), jnp.int32),
            pltpu.VMEM((s2,), jnp.int32),
            pltpu.VMEM((16,), jnp.int32),
            pltpu.VMEM((16,), jnp.int32),
            pltpu.VMEM((rp,), jnp.int32),
            pltpu.VMEM((s2,), jnp.int32),
            pltpu.VMEM((32,), jnp.int32),
        ],
    )
    def route(idx_hbm, rank_hbm, cnt_hbm, stid_hbm, pos_hbm, be_hbm,
              idx_v, rank_v, cnt_v, cb_v, stid_v, pos_v, be_v):
        wid = lax.axis_index("s") * 2 + lax.axis_index("c")

        @pl.when(wid == 0)
        def _():
            pltpu.sync_copy(idx_hbm, idx_v)
            pltpu.sync_copy(rank_hbm, rank_v)
            pltpu.sync_copy(cnt_hbm, cnt_v)
            c = cnt_v[...]
            nb = lax.shift_right_logical(c + (mblk - 1), shift)
            cbi = plsc.cumsum(nb)
            cb = cbi - nb
            cb_v[...] = cb
            be_v[pl.ds(0, 16)] = jnp.zeros((16,), jnp.int32)
            be_v[pl.ds(16, 16)] = jnp.zeros((16,), jnp.int32)
            lane = lax.iota(jnp.int32, 16)
            emask = jnp.logical_and(lane >= 1, lane < n_exp)
            plsc.addupdate_scatter(
                be_v, [cb], jnp.ones((16,), jnp.int32), mask=emask)
            d0 = be_v[pl.ds(0, 16)]
            d1 = be_v[pl.ds(16, 16)]
            be_v[pl.ds(0, 16)] = plsc.cumsum(d0)
            be_v[pl.ds(16, 16)] = plsc.cumsum(d1) + jnp.sum(d0)

            def init(i, _):
                stid_v[pl.ds(i * 16, 16)] = jnp.zeros((16,), jnp.int32)
                return 0

            lax.fori_loop(0, rp // 16, init, 0)
            tmask = jnp.int32(s2 // 2 - 1)

            def body(i, _):
                a0 = i * 16
                ev = idx_v[pl.ds(a0, 16)]
                rk = rank_v[pl.ds(a0, 16)]
                cbg = plsc.load_gather(cb_v, [ev])
                pos = cbg * mblk + rk
                tok = lax.bitwise_and(a0 + lax.iota(jnp.int32, 16), tmask)
                plsc.store_scatter(stid_v, [pos], tok)
                pos_v[pl.ds(a0, 16)] = pos
                return 0

            lax.fori_loop(0, s2 // 16, body, 0)
            pltpu.sync_copy(stid_v, stid_hbm)
            pltpu.sync_copy(pos_v, pos_hbm)
            pltpu.sync_copy(be_v, be_hbm)

    return route(idx_flat, rank_flat, counts16)


def _sc_gather(table, idx):
    bn = idx.shape[0]
    dd = table.shape[1]
    nw = 32
    bpw = bn // nw
    ch = bpw if bpw <= 64 else 64
    nch = bpw // ch
    mesh = plsc.VectorSubcoreMesh(core_axis_name="c", subcore_axis_name="s")

    @functools.partial(
        pl.kernel,
        out_type=jax.ShapeDtypeStruct((bn, dd), jnp.float32),
        mesh=mesh,
        compiler_params=pltpu.CompilerParams(needs_layout_passes=False),
        scratch_types=[
            pltpu.VMEM((ch,), jnp.int32),
            pltpu.VMEM((ch, dd), jnp.float32),
            pltpu.SemaphoreType.DMA,
        ],
    )
    def gather(table_hbm, idx_hbm, out_hbm, idx_v, rows_v, sem):
        wid = lax.axis_index("s") * 2 + lax.axis_index("c")
        base = wid * bpw

        def body(i, _):
            b0 = base + i * ch
            pltpu.sync_copy(idx_hbm.at[pl.ds(b0, ch)], idx_v)
            pltpu.async_copy(table_hbm.at[idx_v], rows_v, sem).wait()
            pltpu.sync_copy(rows_v, out_hbm.at[pl.ds(b0, ch)])
            return 0

        lax.fori_loop(0, nch, body, 0)

    return gather(table, idx)


def _encoder_layer(x, w3, b3, wo3, bo, g1, be1, g2, be2, wg_t, bg,
                   w1_t, b1, w2_t, b2):
    S, D = x.shape
    E, _, DFF = w1_t.shape
    K = 2
    hd = D // H
    sblk = 256 if S % 256 == 0 else S
    nsb = S // sblk
    rp = K * S + E * MBLK
    nblk = rp // MBLK

    qkv = pl.pallas_call(
        _qkv_proj_kernel,
        grid=(nsb,),
        in_specs=[
            pl.BlockSpec((sblk, D), lambda i: (i, 0)),
            pl.BlockSpec((3 * H, D, hd), lambda i: (0, 0, 0)),
            pl.BlockSpec((3 * H, 1, hd), lambda i: (0, 0, 0)),
        ],
        out_specs=pl.BlockSpec((3 * H, sblk, hd), lambda i: (0, i, 0)),
        out_shape=jax.ShapeDtypeStruct((3 * H, S, hd), jnp.float32),
        interpret=_INTERPRET,
    )(x, w3, b3)

    attn = pl.pallas_call(
        functools.partial(_attn_kernel, scale=1.0 / float(hd) ** 0.5),
        grid=(H, nsb),
        in_specs=[
            pl.BlockSpec((1, sblk, hd), lambda h, i: (h, i, 0)),
            pl.BlockSpec((1, S, hd), lambda h, i: (H + h, 0, 0)),
            pl.BlockSpec((1, S, hd), lambda h, i: (2 * H + h, 0, 0)),
        ],
        out_specs=pl.BlockSpec((1, sblk, hd), lambda h, i: (h, i, 0)),
        out_shape=jax.ShapeDtypeStruct((H, S, hd), jnp.float32),
        interpret=_INTERPRET,
    )(qkv, qkv, qkv)

    x1 = pl.pallas_call(
        _oproj_ln_kernel,
        grid=(nsb,),
        in_specs=[
            pl.BlockSpec((H, sblk, hd), lambda i: (0, i, 0)),
            pl.BlockSpec((H, hd, D), lambda i: (0, 0, 0)),
            pl.BlockSpec((1, D), lambda i: (0, 0)),
            pl.BlockSpec((sblk, D), lambda i: (i, 0)),
            pl.BlockSpec((1, D), lambda i: (0, 0)),
            pl.BlockSpec((1, D), lambda i: (0, 0)),
        ],
        out_specs=pl.BlockSpec((sblk, D), lambda i: (i, 0)),
        out_shape=jax.ShapeDtypeStruct((S, D), jnp.float32),
        interpret=_INTERPRET,
    )(attn, wo3, bo.reshape(1, D), x, g1.reshape(1, D), be1.reshape(1, D))

    probs, idx2, usage = pl.pallas_call(
        functools.partial(_router_kernel, n_exp=E),
        grid=(nsb,),
        in_specs=[
            pl.BlockSpec((sblk, D), lambda i: (i, 0)),
            pl.BlockSpec((D, E), lambda i: (0, 0)),
            pl.BlockSpec((1, E), lambda i: (0, 0)),
        ],
        out_specs=[
            pl.BlockSpec((sblk, K), lambda i: (i, 0)),
            pl.BlockSpec((K, sblk, 1), lambda i: (0, i, 0)),
            pl.BlockSpec((1, E), lambda i: (0, 0)),
        ],
        out_shape=[
            jax.ShapeDtypeStruct((S, K), jnp.float32),
            jax.ShapeDtypeStruct((K, S, 1), jnp.int32),
            jax.ShapeDtypeStruct((1, E), jnp.float32),
        ],
        interpret=_INTERPRET,
    )(x1, wg_t, bg.reshape(1, E))

    rank2, counts = pl.pallas_call(
        functools.partial(_rank_kernel, n_exp=E, nrb=nsb, sblk=sblk),
        grid=(K, nsb),
        in_specs=[pl.BlockSpec((1, sblk, 1), lambda k, i: (k, i, 0))],
        out_specs=[
            pl.BlockSpec((1, sblk, 1), lambda k, i: (k, i, 0)),
            pl.BlockSpec((1, 16), lambda k, i: (0, 0)),
        ],
        out_shape=[
            jax.ShapeDtypeStruct((K, S, 1), jnp.int32),
            jax.ShapeDtypeStruct((1, 16), jnp.int32),
        ],
        scratch_shapes=[pltpu.VMEM((1, E), jnp.float32)],
        interpret=_INTERPRET,
    )(idx2)

    stid, pos, be32 = _sc_route(
        idx2.reshape(K * S), rank2.reshape(K * S), counts.reshape(16),
        E, rp, MBLK)

    xs = _sc_gather(x1, stid)

    ys = pl.pallas_call(
        _group_ffn_kernel,
        grid_spec=pltpu.PrefetchScalarGridSpec(
            num_scalar_prefetch=1,
            grid=(nblk,),
            in_specs=[
                pl.BlockSpec((MBLK, D), lambda s, be: (s, 0)),
                pl.BlockSpec((1, D, DFF), lambda s, be: (be[s], 0, 0)),
                pl.BlockSpec((1, 1, DFF), lambda s, be: (be[s], 0, 0)),
                pl.BlockSpec((1, DFF, D), lambda s, be: (be[s], 0, 0)),
                pl.BlockSpec((1, 1, D), lambda s, be: (be[s], 0, 0)),
            ],
            out_specs=pl.BlockSpec((MBLK, D), lambda s, be: (s, 0)),
        ),
        out_shape=jax.ShapeDtypeStruct((rp, D), jnp.float32),
        interpret=_INTERPRET,
    )(be32, xs, w1_t, b1.reshape(E, 1, DFF), w2_t, b2.reshape(E, 1, D))

    ysg = _sc_gather(ys, pos)

    x2 = pl.pallas_call(
        _combine_ln_kernel,
        grid=(nsb,),
        in_specs=[
            pl.BlockSpec((sblk, D), lambda i: (i, 0)),
            pl.BlockSpec((K, sblk, D), lambda i: (0, i, 0)),
            pl.BlockSpec((sblk, K), lambda i: (i, 0)),
            pl.BlockSpec((1, D), lambda i: (0, 0)),
            pl.BlockSpec((1, D), lambda i: (0, 0)),
        ],
        out_specs=pl.BlockSpec((sblk, D), lambda i: (i, 0)),
        out_shape=jax.ShapeDtypeStruct((S, D), jnp.float32),
        interpret=_INTERPRET,
    )(x1, ysg.reshape(K, S, D), probs, g2.reshape(1, D), be2.reshape(1, D))

    return x2, usage


def kernel(src, Wqkv, bqkv, Wo, bo, g1, be1, g2, be2, Wg, bg, W1, b1, W2, b2):
    L = Wqkv.shape[0]
    S, B, D = src.shape
    hd = D // H
    x = src.reshape(S * B, D)
    usages = []
    for l in range(L):
        x, usage = _encoder_layer(
            x,
            jnp.swapaxes(Wqkv[l].reshape(3 * H, hd, D), 1, 2),
            bqkv[l].reshape(3 * H, 1, hd),
            Wo[l].T.reshape(H, hd, D),
            bo[l], g1[l], be1[l], g2[l], be2[l],
            Wg[l].T, bg[l], jnp.swapaxes(W1[l], 1, 2), b1[l],
            jnp.swapaxes(W2[l], 1, 2), b2[l],
        )
        usages.append(usage)

    usage_all = jnp.concatenate(usages, axis=0)
    aux = pl.pallas_call(
        functools.partial(_aux_kernel, n_layers=L),
        out_shape=jax.ShapeDtypeStruct((1, 1), jnp.float32),
        interpret=_INTERPRET,
    )(usage_all)
    return x.reshape(S, B, D), aux.reshape(())
